# baseline (speedup 1.0000x reference)
import functools

import jax
import jax.numpy as jnp
from jax import lax
from jax.experimental import pallas as pl
from jax.experimental.pallas import tpu as pltpu
from jax.experimental.pallas import tpu_sc as plsc

K = 20
N = 1024
B = 32
NEG = -3.0e38
_f32 = jnp.float32


def _lrelu(x):
    return jnp.where(x >= 0, x, 0.2 * x)


def _dot_bf16(a, b):
    return lax.dot_general(a.astype(jnp.bfloat16), b.astype(jnp.bfloat16),
                           (((1,), (0,)), ((), ())),
                           preferred_element_type=_f32)


TB = 2


def _topk_body(xt_ref, xp_ref, out_ref):
    xb = xt_ref[...]
    g2 = 2.0 * lax.dot_general(xb, xb, (((1,), (1,)), ((0,), (0,))),
                               preferred_element_type=_f32)
    xp = xp_ref[...]
    sqc = jnp.sum(xp * xp, axis=2, keepdims=True)
    sqr = jnp.sum(xb * xb, axis=1, keepdims=True)
    d = (g2 - sqc) - sqr
    rid = lax.broadcasted_iota(jnp.int32, (TB, N, N), 1)
    base = (pl.program_id(0) * TB
            + lax.broadcasted_iota(jnp.int32, (TB, 1, N), 0)) * N
    for k in range(K):
        m = jnp.max(d, axis=1, keepdims=True)
        cand = jnp.where(d == m, rid, N)
        amin = jnp.min(cand, axis=1, keepdims=True)
        out_ref[:, pl.ds(k, 1), :] = amin + base
        d = jnp.where(rid == amin, NEG, d)


def _sc_gather(table, idx_flat):
    info = plsc.get_sparse_core_info()
    nw = info.num_cores * info.num_subcores
    total = idx_flat.shape[0]
    per_w = total // nw
    assert per_w * nw == total
    chunk = 2048
    while per_w % chunk:
        chunk //= 2
    iters = per_w // chunk
    nbuf = 2
    nrows = table.shape[0]
    mesh = plsc.VectorSubcoreMesh(core_axis_name="c", subcore_axis_name="s")

    @functools.partial(
        pl.kernel, mesh=mesh,
        compiler_params=pltpu.CompilerParams(use_tc_tiling_on_sc=False),
        out_type=jax.ShapeDtypeStruct((total, 16), _f32),
        scratch_types=[
            pltpu.VMEM_SHARED((nrows, 16), _f32),
            pltpu.VMEM((nbuf, chunk), jnp.int32),
            pltpu.VMEM((nbuf, chunk, 16), _f32),
            pltpu.SemaphoreType.DMA,
            pltpu.SemaphoreType.DMA,
            pltpu.SemaphoreType.DMA,
            pltpu.SemaphoreType.DMA,
            pltpu.SemaphoreType.DMA,
        ],
    )
    def gk(table_hbm, idx_hbm, out_hbm, shared, idx_v, rows_v,
           sb0, sb1, sc0, sc1, st):
        sid = lax.axis_index("s")
        wid = sid * info.num_cores + lax.axis_index("c")
        base = wid * per_w

        @pl.when(sid == 0)
        def _():
            pltpu.async_copy(table_hbm, shared, st).wait()
        plsc.subcore_barrier()

        sb = (sb0, sb1)
        sc = (sc0, sc1)
        bh = [None] * iters
        ch = [None] * iters
        for i in range(iters):
            p = i % nbuf
            if i >= nbuf:
                ch[i - nbuf].wait()
            pltpu.sync_copy(idx_hbm.at[pl.ds(base + i * chunk, chunk)],
                            idx_v.at[p])
            bh[i] = pltpu.async_copy(shared.at[idx_v.at[p]], rows_v.at[p],
                                     sb[p])
            if i >= 1:
                q = (i - 1) % nbuf
                bh[i - 1].wait()
                ch[i - 1] = pltpu.async_copy(
                    rows_v.at[q],
                    out_hbm.at[pl.ds(base + (i - 1) * chunk, chunk)], sc[q])
        q = (iters - 1) % nbuf
        bh[iters - 1].wait()
        ch[iters - 1] = pltpu.async_copy(
            rows_v.at[q], out_hbm.at[pl.ds(base + (iters - 1) * chunk, chunk)],
            sc[q])
        ch[iters - 2].wait()
        ch[iters - 1].wait()

    return gk(table, idx_flat)


def _edge_body(g_ref, xp_ref, w0_ref, b0_ref, m0_ref, s_ref, ss_ref):
    nb = g_ref[0].reshape(K, N, 16)
    xp = xp_ref[0]
    feat = jnp.concatenate(
        [nb - xp[None], jnp.broadcast_to(xp[None], (K, N, 16))], axis=-1)
    t = _dot_bf16(feat.reshape(K * N, 32), w0_ref[...]) + b0_ref[...]
    t = t.reshape(K, N, 64)
    m0_ref[0] = jnp.max(t, axis=0)
    s_ref[0] = jnp.sum(jnp.sum(t, axis=0), axis=0, keepdims=True)
    ss_ref[0] = jnp.sum(jnp.sum(t * t, axis=0), axis=0, keepdims=True)


def _pconv_body(h_ref, sc_ref, sh_ref, w_ref, b_ref, o_ref, s_ref, ss_ref):
    h = _lrelu(h_ref[0] * sc_ref[...] + sh_ref[...])
    g = _dot_bf16(h, w_ref[...]) + b_ref[...]
    o_ref[0] = g
    s_ref[0] = jnp.sum(g, axis=0, keepdims=True)
    ss_ref[0] = jnp.sum(g * g, axis=0, keepdims=True)


def _pconv_max_body(h_ref, sc_ref, sh_ref, w_ref, b_ref, mx_ref, s_ref, ss_ref):
    h = _lrelu(h_ref[0] * sc_ref[...] + sh_ref[...])
    g = _dot_bf16(h, w_ref[...]) + b_ref[...]
    mx_ref[0] = jnp.max(g, axis=0, keepdims=True)
    s_ref[0] = jnp.sum(g, axis=0, keepdims=True)
    ss_ref[0] = jnp.sum(g * g, axis=0, keepdims=True)


def _head_body(m3_ref, sc3_ref, sh3_ref, w4_ref, b4_ref, g4_ref, be4_ref,
               w5_ref, b5_ref, g5_ref, be5_ref, w6_ref, b6_ref, o_ref):
    h = _lrelu(m3_ref[...] * sc3_ref[...] + sh3_ref[...])
    for w_r, b_r, g_r, be_r in ((w4_ref, b4_ref, g4_ref, be4_ref),
                                (w5_ref, b5_ref, g5_ref, be5_ref)):
        u = _dot_bf16(h, w_r[...]) + b_r[...]
        mu = jnp.mean(u, axis=0, keepdims=True)
        va = jnp.mean(u * u, axis=0, keepdims=True) - mu * mu
        h = _lrelu((u - mu) / jnp.sqrt(va + 1e-5) * g_r[...] + be_r[...])
    o_ref[...] = _dot_bf16(h, w6_ref[...]) + b6_ref[...]


def _finish_stats(s, ss, cnt, g, be):
    mu = jnp.sum(s, axis=0) / cnt
    va = jnp.sum(ss, axis=0) / cnt - mu * mu
    scale = g.reshape(1, -1) / jnp.sqrt(va + 1e-5)
    shift = be.reshape(1, -1) - mu * scale
    return scale, shift


def _bspec(shape, const=False):
    nd = len(shape)
    if const:
        return pl.BlockSpec(shape, lambda b: (0,) * nd)
    return pl.BlockSpec(shape, lambda b: (b,) + (0,) * (nd - 1))


def _pconv_call(h, scale, shift, wt, bias, cout):
    cin = h.shape[-1]
    return pl.pallas_call(
        _pconv_body, grid=(B,),
        in_specs=[_bspec((1, N, cin)), _bspec((1, cin), True),
                  _bspec((1, cin), True), _bspec((cin, cout), True),
                  _bspec((1, cout), True)],
        out_specs=[_bspec((1, N, cout)), _bspec((1, 1, cout)),
                   _bspec((1, 1, cout))],
        out_shape=[jax.ShapeDtypeStruct((B, N, cout), _f32),
                   jax.ShapeDtypeStruct((B, 1, cout), _f32),
                   jax.ShapeDtypeStruct((B, 1, cout), _f32)],
    )(h, scale, shift, wt, bias.reshape(1, cout))


def kernel(x, params):
    p = params
    x = x.astype(_f32)
    xt = jnp.transpose(x, (0, 2, 1))
    xpad = jnp.concatenate(
        [x, jnp.zeros((B, N, 13), _f32)], axis=-1)

    idx = pl.pallas_call(
        _topk_body, grid=(B // TB,),
        in_specs=[_bspec((TB, 3, N)), _bspec((TB, N, 16))],
        out_specs=_bspec((TB, K, N)),
        out_shape=jax.ShapeDtypeStruct((B, K, N), jnp.int32),
    )(xt, xpad)

    gathered = _sc_gather(xpad.reshape(B * N, 16), idx.reshape(B * K * N))
    gath = gathered.reshape(B, K * N, 16)

    w0 = p['W0']
    zpad = jnp.zeros((13, 64), _f32)
    w0p = jnp.concatenate([w0[:, :3].T, zpad, w0[:, 3:].T, zpad], axis=0)

    m0, s0, ss0 = pl.pallas_call(
        _edge_body, grid=(B,),
        in_specs=[_bspec((1, K * N, 16)), _bspec((1, N, 16)),
                  _bspec((32, 64), True),
                  _bspec((1, 64), True)],
        out_specs=[_bspec((1, N, 64)), _bspec((1, 1, 64)), _bspec((1, 1, 64))],
        out_shape=[jax.ShapeDtypeStruct((B, N, 64), _f32),
                   jax.ShapeDtypeStruct((B, 1, 64), _f32),
                   jax.ShapeDtypeStruct((B, 1, 64), _f32)],
    )(gath, xpad, w0p, p['b0'].reshape(1, 64))

    sc0, sh0 = _finish_stats(s0, ss0, B * N * K, p['g0'], p['be0'])
    g1, s1, ss1 = _pconv_call(m0, sc0, sh0, p['W1'].T, p['b1'], 128)
    sc1, sh1 = _finish_stats(s1, ss1, B * N, p['g1'], p['be1'])
    g2, s2, ss2 = _pconv_call(g1, sc1, sh1, p['W2'].T, p['b2'], 128)
    sc2, sh2 = _finish_stats(s2, ss2, B * N, p['g2'], p['be2'])

    m3, s3, ss3 = pl.pallas_call(
        _pconv_max_body, grid=(B,),
        in_specs=[_bspec((1, N, 128)), _bspec((1, 128), True),
                  _bspec((1, 128), True), _bspec((128, 1024), True),
                  _bspec((1, 1024), True)],
        out_specs=[_bspec((1, 1, 1024)), _bspec((1, 1, 1024)),
                   _bspec((1, 1, 1024))],
        out_shape=[jax.ShapeDtypeStruct((B, 1, 1024), _f32),
                   jax.ShapeDtypeStruct((B, 1, 1024), _f32),
                   jax.ShapeDtypeStruct((B, 1, 1024), _f32)],
    )(g2, sc2, sh2, p['W3'].T, p['b3'].reshape(1, 1024))

    sc3, sh3 = _finish_stats(s3, ss3, B * N, p['g3'], p['be3'])

    out = pl.pallas_call(
        _head_body,
        out_shape=jax.ShapeDtypeStruct((B, 2 * 512), _f32),
    )(m3.reshape(B, 1024), sc3, sh3,
      p['W4'].T, p['b4'].reshape(1, 512), p['g4'].reshape(1, 512),
      p['be4'].reshape(1, 512),
      p['W5'].T, p['b5'].reshape(1, 512), p['g5'].reshape(1, 512),
      p['be5'].reshape(1, 512),
      p['W6'].T, p['b6'].reshape(1, 1024))
    return out


# speedup vs baseline: 1.2756x; 1.2756x over previous
import functools

import jax
import jax.numpy as jnp
from jax import lax
from jax.experimental import pallas as pl
from jax.experimental.pallas import tpu as pltpu
from jax.experimental.pallas import tpu_sc as plsc

K = 20
N = 1024
B = 32
NEG = -3.0e38
_f32 = jnp.float32


def _lrelu(x):
    return jnp.where(x >= 0, x, 0.2 * x)


def _dot_bf16(a, b):
    return lax.dot_general(a.astype(jnp.bfloat16), b.astype(jnp.bfloat16),
                           (((1,), (0,)), ((), ())),
                           preferred_element_type=_f32)


TB = 1


def _topk_body(xt_ref, xp_ref, out_ref):
    xb = xt_ref[0]
    g2 = 2.0 * lax.dot_general(xb, xb, (((0,), (0,)), ((), ())),
                               preferred_element_type=_f32)
    xp = xp_ref[0]
    sqc = jnp.sum(xp * xp, axis=1, keepdims=True)
    sqr = jnp.sum(xb * xb, axis=0, keepdims=True)
    d = (g2 - sqc) - sqr
    rid = lax.broadcasted_iota(jnp.int32, (N, N), 0)
    base = pl.program_id(0) * N
    for k in range(K):
        am = lax.argmax(d, 0, jnp.int32).reshape(1, N)
        out_ref[0, pl.ds(k, 1), :] = am + base
        d = jnp.where(rid == am, NEG, d)


def _sc_gather(table, idx_flat):
    info = plsc.get_sparse_core_info()
    nw = info.num_cores * info.num_subcores
    total = idx_flat.shape[0]
    per_w = total // nw
    assert per_w * nw == total
    chunk = 2048
    while per_w % chunk:
        chunk //= 2
    iters = per_w // chunk
    nbuf = 2
    nrows = table.shape[0]
    mesh = plsc.VectorSubcoreMesh(core_axis_name="c", subcore_axis_name="s")

    @functools.partial(
        pl.kernel, mesh=mesh,
        compiler_params=pltpu.CompilerParams(use_tc_tiling_on_sc=False),
        out_type=jax.ShapeDtypeStruct((total, 16), _f32),
        scratch_types=[
            pltpu.VMEM_SHARED((nrows, 16), _f32),
            pltpu.VMEM((nbuf, chunk), jnp.int32),
            pltpu.VMEM((nbuf, chunk, 16), _f32),
            pltpu.SemaphoreType.DMA,
            pltpu.SemaphoreType.DMA,
            pltpu.SemaphoreType.DMA,
            pltpu.SemaphoreType.DMA,
            pltpu.SemaphoreType.DMA,
        ],
    )
    def gk(table_hbm, idx_hbm, out_hbm, shared, idx_v, rows_v,
           sb0, sb1, sc0, sc1, st):
        sid = lax.axis_index("s")
        wid = sid * info.num_cores + lax.axis_index("c")
        base = wid * per_w

        @pl.when(sid == 0)
        def _():
            pltpu.async_copy(table_hbm, shared, st).wait()
        plsc.subcore_barrier()

        sb = (sb0, sb1)
        sc = (sc0, sc1)
        bh = [None] * iters
        ch = [None] * iters
        for i in range(iters):
            p = i % nbuf
            if i >= nbuf:
                ch[i - nbuf].wait()
            pltpu.sync_copy(idx_hbm.at[pl.ds(base + i * chunk, chunk)],
                            idx_v.at[p])
            bh[i] = pltpu.async_copy(shared.at[idx_v.at[p]], rows_v.at[p],
                                     sb[p])
            if i >= 1:
                q = (i - 1) % nbuf
                bh[i - 1].wait()
                ch[i - 1] = pltpu.async_copy(
                    rows_v.at[q],
                    out_hbm.at[pl.ds(base + (i - 1) * chunk, chunk)], sc[q])
        q = (iters - 1) % nbuf
        bh[iters - 1].wait()
        ch[iters - 1] = pltpu.async_copy(
            rows_v.at[q], out_hbm.at[pl.ds(base + (iters - 1) * chunk, chunk)],
            sc[q])
        ch[iters - 2].wait()
        ch[iters - 1].wait()

    return gk(table, idx_flat)


def _edge_body(g_ref, xp_ref, w0_ref, b0_ref, m0_ref, s_ref, ss_ref):
    nb = g_ref[0].reshape(K, N, 16)
    xp = xp_ref[0]
    feat = jnp.concatenate(
        [nb - xp[None], jnp.broadcast_to(xp[None], (K, N, 16))], axis=-1)
    t = _dot_bf16(feat.reshape(K * N, 32), w0_ref[...]) + b0_ref[...]
    t = t.reshape(K, N, 64)
    m0_ref[0] = jnp.max(t, axis=0)
    s_ref[0] = jnp.sum(jnp.sum(t, axis=0), axis=0, keepdims=True)
    ss_ref[0] = jnp.sum(jnp.sum(t * t, axis=0), axis=0, keepdims=True)


def _pconv_body(h_ref, sc_ref, sh_ref, w_ref, b_ref, o_ref, s_ref, ss_ref):
    h = _lrelu(h_ref[0] * sc_ref[...] + sh_ref[...])
    g = _dot_bf16(h, w_ref[...]) + b_ref[...]
    o_ref[0] = g
    s_ref[0] = jnp.sum(g, axis=0, keepdims=True)
    ss_ref[0] = jnp.sum(g * g, axis=0, keepdims=True)


def _pconv_max_body(h_ref, sc_ref, sh_ref, w_ref, b_ref, mx_ref, s_ref, ss_ref):
    h = _lrelu(h_ref[0] * sc_ref[...] + sh_ref[...])
    g = _dot_bf16(h, w_ref[...]) + b_ref[...]
    mx_ref[0] = jnp.max(g, axis=0, keepdims=True)
    s_ref[0] = jnp.sum(g, axis=0, keepdims=True)
    ss_ref[0] = jnp.sum(g * g, axis=0, keepdims=True)


def _head_body(m3_ref, sc3_ref, sh3_ref, w4_ref, b4_ref, g4_ref, be4_ref,
               w5_ref, b5_ref, g5_ref, be5_ref, w6_ref, b6_ref, o_ref):
    h = _lrelu(m3_ref[...] * sc3_ref[...] + sh3_ref[...])
    for w_r, b_r, g_r, be_r in ((w4_ref, b4_ref, g4_ref, be4_ref),
                                (w5_ref, b5_ref, g5_ref, be5_ref)):
        u = _dot_bf16(h, w_r[...]) + b_r[...]
        mu = jnp.mean(u, axis=0, keepdims=True)
        va = jnp.mean(u * u, axis=0, keepdims=True) - mu * mu
        h = _lrelu((u - mu) / jnp.sqrt(va + 1e-5) * g_r[...] + be_r[...])
    o_ref[...] = _dot_bf16(h, w6_ref[...]) + b6_ref[...]


def _finish_stats(s, ss, cnt, g, be):
    mu = jnp.sum(s, axis=0) / cnt
    va = jnp.sum(ss, axis=0) / cnt - mu * mu
    scale = g.reshape(1, -1) / jnp.sqrt(va + 1e-5)
    shift = be.reshape(1, -1) - mu * scale
    return scale, shift


def _bspec(shape, const=False):
    nd = len(shape)
    if const:
        return pl.BlockSpec(shape, lambda b: (0,) * nd)
    return pl.BlockSpec(shape, lambda b: (b,) + (0,) * (nd - 1))


def _pconv_call(h, scale, shift, wt, bias, cout):
    cin = h.shape[-1]
    return pl.pallas_call(
        _pconv_body, grid=(B,),
        in_specs=[_bspec((1, N, cin)), _bspec((1, cin), True),
                  _bspec((1, cin), True), _bspec((cin, cout), True),
                  _bspec((1, cout), True)],
        out_specs=[_bspec((1, N, cout)), _bspec((1, 1, cout)),
                   _bspec((1, 1, cout))],
        out_shape=[jax.ShapeDtypeStruct((B, N, cout), _f32),
                   jax.ShapeDtypeStruct((B, 1, cout), _f32),
                   jax.ShapeDtypeStruct((B, 1, cout), _f32)],
    )(h, scale, shift, wt, bias.reshape(1, cout))


def kernel(x, params):
    p = params
    x = x.astype(_f32)
    xt = jnp.transpose(x, (0, 2, 1))
    xpad = jnp.concatenate(
        [x, jnp.zeros((B, N, 13), _f32)], axis=-1)

    idx = pl.pallas_call(
        _topk_body, grid=(B,),
        in_specs=[_bspec((1, 3, N)), _bspec((1, N, 16))],
        out_specs=_bspec((1, K, N)),
        out_shape=jax.ShapeDtypeStruct((B, K, N), jnp.int32),
    )(xt, xpad)

    gathered = _sc_gather(xpad.reshape(B * N, 16), idx.reshape(B * K * N))
    gath = gathered.reshape(B, K * N, 16)

    w0 = p['W0']
    zpad = jnp.zeros((13, 64), _f32)
    w0p = jnp.concatenate([w0[:, :3].T, zpad, w0[:, 3:].T, zpad], axis=0)

    m0, s0, ss0 = pl.pallas_call(
        _edge_body, grid=(B,),
        in_specs=[_bspec((1, K * N, 16)), _bspec((1, N, 16)),
                  _bspec((32, 64), True),
                  _bspec((1, 64), True)],
        out_specs=[_bspec((1, N, 64)), _bspec((1, 1, 64)), _bspec((1, 1, 64))],
        out_shape=[jax.ShapeDtypeStruct((B, N, 64), _f32),
                   jax.ShapeDtypeStruct((B, 1, 64), _f32),
                   jax.ShapeDtypeStruct((B, 1, 64), _f32)],
    )(gath, xpad, w0p, p['b0'].reshape(1, 64))

    sc0, sh0 = _finish_stats(s0, ss0, B * N * K, p['g0'], p['be0'])
    g1, s1, ss1 = _pconv_call(m0, sc0, sh0, p['W1'].T, p['b1'], 128)
    sc1, sh1 = _finish_stats(s1, ss1, B * N, p['g1'], p['be1'])
    g2, s2, ss2 = _pconv_call(g1, sc1, sh1, p['W2'].T, p['b2'], 128)
    sc2, sh2 = _finish_stats(s2, ss2, B * N, p['g2'], p['be2'])

    m3, s3, ss3 = pl.pallas_call(
        _pconv_max_body, grid=(B,),
        in_specs=[_bspec((1, N, 128)), _bspec((1, 128), True),
                  _bspec((1, 128), True), _bspec((128, 1024), True),
                  _bspec((1, 1024), True)],
        out_specs=[_bspec((1, 1, 1024)), _bspec((1, 1, 1024)),
                   _bspec((1, 1, 1024))],
        out_shape=[jax.ShapeDtypeStruct((B, 1, 1024), _f32),
                   jax.ShapeDtypeStruct((B, 1, 1024), _f32),
                   jax.ShapeDtypeStruct((B, 1, 1024), _f32)],
    )(g2, sc2, sh2, p['W3'].T, p['b3'].reshape(1, 1024))

    sc3, sh3 = _finish_stats(s3, ss3, B * N, p['g3'], p['be3'])

    out = pl.pallas_call(
        _head_body,
        out_shape=jax.ShapeDtypeStruct((B, 2 * 512), _f32),
    )(m3.reshape(B, 1024), sc3, sh3,
      p['W4'].T, p['b4'].reshape(1, 512), p['g4'].reshape(1, 512),
      p['be4'].reshape(1, 512),
      p['W5'].T, p['b5'].reshape(1, 512), p['g5'].reshape(1, 512),
      p['be5'].reshape(1, 512),
      p['W6'].T, p['b6'].reshape(1, 1024))
    return out
